# baseline (speedup 1.0000x reference)
import functools

import jax
import jax.numpy as jnp
from jax import lax
from jax.experimental import pallas as pl
from jax.experimental.pallas import tpu as pltpu
from jax.experimental.pallas import tpu_sc as plsc

_NC = plsc.get_sparse_core_info().num_cores

_MESH = plsc.VectorSubcoreMesh(core_axis_name="c", subcore_axis_name="s")

_OUT_T = tuple(jax.ShapeDtypeStruct((128,), jnp.int32) for _ in range(3))


@functools.partial(pl.kernel, mesh=_MESH, out_type=_OUT_T)
def _sc_copy(w_in, p_in, t_in, w_out, p_out, t_out):
    wid = lax.axis_index("s") * _NC + lax.axis_index("c")

    @pl.when(wid == 0)
    def _():
        pltpu.sync_copy(w_in, w_out)

    @pl.when(wid == 1)
    def _():
        pltpu.sync_copy(p_in, p_out)

    @pl.when(wid == 2)
    def _():
        pltpu.sync_copy(t_in, t_out)


def kernel(word_idx_tensor, pos_idx_tensor, true_tree_heads):
    return _sc_copy(word_idx_tensor, pos_idx_tensor, true_tree_heads)


# speedup vs baseline: 1.0217x; 1.0217x over previous
import functools

import jax
import jax.numpy as jnp
from jax import lax
from jax.experimental import pallas as pl
from jax.experimental.pallas import tpu as pltpu
from jax.experimental.pallas import tpu_sc as plsc

_MESH = plsc.ScalarSubcoreMesh(axis_name="c", num_cores=2)

_OUT_T = tuple(jax.ShapeDtypeStruct((128,), jnp.int32) for _ in range(3))


@functools.partial(pl.kernel, mesh=_MESH, out_type=_OUT_T)
def _sc_copy(w_in, p_in, t_in, w_out, p_out, t_out):
    cid = lax.axis_index("c")

    @pl.when(cid == 0)
    def _():
        pltpu.sync_copy(w_in, w_out)
        pltpu.sync_copy(p_in, p_out)

    @pl.when(cid == 1)
    def _():
        pltpu.sync_copy(t_in, t_out)


def kernel(word_idx_tensor, pos_idx_tensor, true_tree_heads):
    return _sc_copy(word_idx_tensor, pos_idx_tensor, true_tree_heads)


# speedup vs baseline: 11.0557x; 10.8207x over previous
import jax
import jax.numpy as jnp
from jax.experimental import pallas as pl


def _copy_body(w_ref, p_ref, t_ref, wo_ref, po_ref, to_ref):
    wo_ref[...] = w_ref[...]
    po_ref[...] = p_ref[...]
    to_ref[...] = t_ref[...]


def kernel(word_idx_tensor, pos_idx_tensor, true_tree_heads):
    out_shape = tuple(
        jax.ShapeDtypeStruct(x.shape, x.dtype)
        for x in (word_idx_tensor, pos_idx_tensor, true_tree_heads)
    )
    return pl.pallas_call(_copy_body, out_shape=out_shape)(
        word_idx_tensor, pos_idx_tensor, true_tree_heads
    )


# speedup vs baseline: 11.9442x; 1.0804x over previous
import jax
import jax.numpy as jnp
from jax.experimental import pallas as pl
from jax.experimental.pallas import tpu as pltpu


def _copy_body(w_ref, p_ref, t_ref, wo_ref, po_ref, to_ref, s0, s1, s2):
    c0 = pltpu.make_async_copy(w_ref, wo_ref, s0)
    c1 = pltpu.make_async_copy(p_ref, po_ref, s1)
    c2 = pltpu.make_async_copy(t_ref, to_ref, s2)
    c0.start()
    c1.start()
    c2.start()
    c0.wait()
    c1.wait()
    c2.wait()


def kernel(word_idx_tensor, pos_idx_tensor, true_tree_heads):
    out_shape = tuple(
        jax.ShapeDtypeStruct(x.shape, x.dtype)
        for x in (word_idx_tensor, pos_idx_tensor, true_tree_heads)
    )
    any_spec = pl.BlockSpec(memory_space=pl.ANY)
    return pl.pallas_call(
        _copy_body,
        out_shape=out_shape,
        in_specs=[any_spec] * 3,
        out_specs=[any_spec] * 3,
        scratch_shapes=[pltpu.SemaphoreType.DMA] * 3,
    )(word_idx_tensor, pos_idx_tensor, true_tree_heads)
